# baseline (speedup 1.0000x reference)
import functools

import jax
import jax.numpy as jnp
from jax import lax
from jax.experimental import pallas as pl
from jax.experimental.pallas import tpu as pltpu
from jax.experimental.pallas import tpu_sc as plsc

NC = 2
NS = 16
L = 16
NW = NC * NS
B = 16384
D = 64
BPW = B // NW
CH = 128
NCH = BPW // CH


def _sqrt16(x):
    bits = plsc.bitcast(x, jnp.int32)
    seed = plsc.bitcast(jnp.int32(0x1FBD1DF5) + (bits >> 1), jnp.float32)
    y = seed
    for _ in range(3):
        y = 0.5 * (y + x / y)
    return y


def _sc_body(h2d, r2d, t2d, ent, rel, out,
             hidx, ridx, tidx, hv, rv, tv, ps, sc, *sems):
    c = lax.axis_index("c")
    s = lax.axis_index("s")
    wid = s * NC + c
    base = wid * BPW

    pltpu.sync_copy(h2d.at[pl.ds(NCH * wid, NCH)], hidx)
    pltpu.sync_copy(r2d.at[pl.ds(NCH * wid, NCH)], ridx)
    pltpu.sync_copy(t2d.at[pl.ds(NCH * wid, NCH)], tidx)

    copies = []
    for j in range(NCH):
        copies.append(pltpu.async_copy(
            ent.at[hidx.at[j]], hv.at[pl.ds(j * CH, CH)], sems[3 * j]))
        copies.append(pltpu.async_copy(
            rel.at[ridx.at[j]], rv.at[pl.ds(j * CH, CH)], sems[3 * j + 1]))
        copies.append(pltpu.async_copy(
            ent.at[tidx.at[j]], tv.at[pl.ds(j * CH, CH)], sems[3 * j + 2]))

    iota = lax.iota(jnp.int32, L)

    for j in range(NCH):
        copies[3 * j].wait()
        copies[3 * j + 1].wait()
        copies[3 * j + 2].wait()

        def row_body(r, carry, j=j):
            rr = j * CH + r
            acc = None
            for k in range(D // L):
                hh = hv[rr, pl.ds(k * L, L)]
                re = rv[rr, pl.ds(k * L, L)]
                tt = tv[rr, pl.ds(k * L, L)]
                d = (hh + re) - tt
                acc = d * d if acc is None else acc + d * d
            ps[rr] = acc
            return carry

        lax.fori_loop(0, CH, row_body, 0, unroll=4)

        for bq in range(CH // L):
            rows16 = (j * CH + bq * L) + iota
            acc = jnp.zeros((L,), jnp.float32)
            for k in range(L):
                col = jnp.full((L,), k, jnp.int32)
                acc = acc + plsc.load_gather(ps, [rows16, col])
            sc[pl.ds(j * CH + bq * L, L)] = _sqrt16(acc)

    pltpu.sync_copy(sc, out.at[pl.ds(base, BPW)])


@jax.jit
def kernel(head_ids, relation_ids, tail_ids, entity_table, relation_table):
    h2d = head_ids.astype(jnp.int32).reshape(NW * NCH, CH)
    r2d = relation_ids.astype(jnp.int32).reshape(NW * NCH, CH)
    t2d = tail_ids.astype(jnp.int32).reshape(NW * NCH, CH)

    mesh = plsc.VectorSubcoreMesh(core_axis_name="c", subcore_axis_name="s")
    scratch = [
        pltpu.VMEM((NCH, CH), jnp.int32),
        pltpu.VMEM((NCH, CH), jnp.int32),
        pltpu.VMEM((NCH, CH), jnp.int32),
        pltpu.VMEM((BPW, D), jnp.float32),
        pltpu.VMEM((BPW, D), jnp.float32),
        pltpu.VMEM((BPW, D), jnp.float32),
        pltpu.VMEM((BPW, L), jnp.float32),
        pltpu.VMEM((BPW,), jnp.float32),
    ] + [pltpu.SemaphoreType.DMA] * (3 * NCH)

    run = pl.kernel(
        _sc_body,
        out_type=jax.ShapeDtypeStruct((B,), jnp.float32),
        mesh=mesh,
        scratch_types=scratch,
        compiler_params=pltpu.CompilerParams(
            needs_layout_passes=False, use_tc_tiling_on_sc=False),
    )
    return run(h2d, r2d, t2d, entity_table, relation_table)
